# Initial kernel scaffold; baseline (speedup 1.0000x reference)
import functools

import jax
import jax.numpy as jnp
from jax import lax
from jax.experimental import pallas as pl
from jax.experimental.pallas import tpu as pltpu
from jax.experimental.pallas import tpu_sc as plsc

R = 2 ** 18
F = 48
N = 1000000
C = 400
NCH = N // C
NW = 32
SEG = F // 16


def _body(pts_hbm, cb_hbm, out_hbm, pts_v, idx_v, w_v, feat_v, sem):
    wid = lax.axis_index("s") * 2 + lax.axis_index("c")
    nch_w = (NCH - wid + NW - 1) // NW
    lanes = jnp.arange(16, dtype=jnp.int32)
    zeros16 = jnp.zeros((16,), jnp.int32)
    ones16 = jnp.ones((16,), jnp.int32)
    lim = jnp.float32(R - 1 - 1e-5)
    top = jnp.full((16,), R - 1, jnp.int32)

    def chunk_body(k, carry):
        cid = wid + k * NW
        base = cid * C
        pltpu.sync_copy(pts_hbm.at[pl.ds(base, C)], pts_v)

        def idx_body(i, carry2):
            rows = i * 16 + lanes
            px = plsc.load_gather(pts_v, [rows, zeros16])
            py = plsc.load_gather(pts_v, [rows, ones16])
            x = jnp.maximum(jnp.minimum(px * (R - 1.0), lim), 0.0)
            y = jnp.maximum(jnp.minimum(py * (R - 1.0), lim), 0.0)
            xi = x.astype(jnp.int32)
            yi = y.astype(jnp.int32)
            x2 = jnp.minimum(xi + 1, top)
            y2 = jnp.minimum(yi + 1, top)
            sl = pl.ds(i * 16, 16)
            idx_v[sl] = xi * 2
            idx_v[pl.ds(C + i * 16, 16)] = x2 * 2
            idx_v[pl.ds(2 * C + i * 16, 16)] = yi * 2 + 1
            idx_v[pl.ds(3 * C + i * 16, 16)] = y2 * 2 + 1
            w_v[sl] = x2.astype(jnp.float32) - x
            w_v[pl.ds(C + i * 16, 16)] = x - xi.astype(jnp.float32)
            w_v[pl.ds(2 * C + i * 16, 16)] = y2.astype(jnp.float32) - y
            w_v[pl.ds(3 * C + i * 16, 16)] = y - yi.astype(jnp.float32)
            return carry2

        lax.fori_loop(0, C // 16, idx_body, 0)

        pltpu.async_copy(cb_hbm.at[idx_v], feat_v, sem).wait()

        def comb_body(c, carry2):
            w1 = w_v[c]
            w2 = w_v[C + c]
            w3 = w_v[2 * C + c]
            w4 = w_v[3 * C + c]
            for s in range(SEG):
                sl = pl.ds(s * 16, 16)
                a = feat_v[c, sl]
                b = feat_v[C + c, sl]
                cc = feat_v[2 * C + c, sl]
                d = feat_v[3 * C + c, sl]
                feat_v[c, sl] = (w1 * a + w2 * b) + (w3 * cc + w4 * d)
            return carry2

        lax.fori_loop(0, C, comb_body, 0)

        pltpu.sync_copy(feat_v.at[pl.ds(0, C)], out_hbm.at[pl.ds(base, C)])
        return carry

    lax.fori_loop(0, nch_w, chunk_body, 0)


@jax.jit
def kernel(pts, codebook_0):
    cb2 = codebook_0.reshape(2 * R, F)
    run = functools.partial(
        pl.kernel,
        mesh=plsc.VectorSubcoreMesh(core_axis_name="c", subcore_axis_name="s"),
        out_type=jax.ShapeDtypeStruct((N, F), jnp.float32),
        scratch_types=[
            pltpu.VMEM((C, 2), jnp.float32),
            pltpu.VMEM((4 * C,), jnp.int32),
            pltpu.VMEM((4 * C,), jnp.float32),
            pltpu.VMEM((4 * C, F), jnp.float32),
            pltpu.SemaphoreType.DMA,
        ],
    )(_body)
    return run(pts, cb2)


# speedup vs baseline: 1.1560x; 1.1560x over previous
import functools

import jax
import jax.numpy as jnp
from jax import lax
from jax.experimental import pallas as pl
from jax.experimental.pallas import tpu as pltpu
from jax.experimental.pallas import tpu_sc as plsc

R = 2 ** 18
F = 48
N = 1000000
C = 400
NCH = N // C
NW = 32
SEG = F // 16


def _body(pts_hbm, cb_hbm, out_hbm, pts_v, idx_v, w_v, feat_v, sem):
    wid = lax.axis_index("s") * 2 + lax.axis_index("c")
    nch_w = (NCH - wid + NW - 1) // NW
    lanes = jnp.arange(16, dtype=jnp.int32)
    lim = jnp.float32(R - 1 - 1e-5)
    top = jnp.full((16,), R - 1, jnp.int32)

    def chunk_body(k, carry):
        cid = wid + k * NW
        base = cid * C
        pltpu.sync_copy(pts_hbm.at[pl.ds(2 * base, 2 * C)], pts_v)

        def idx_body(i, carry2):
            rows = (i * 16 + lanes) * 2
            px = plsc.load_gather(pts_v, [rows])
            py = plsc.load_gather(pts_v, [rows + 1])
            x = jnp.maximum(jnp.minimum(px * (R - 1.0), lim), 0.0)
            y = jnp.maximum(jnp.minimum(py * (R - 1.0), lim), 0.0)
            xi = x.astype(jnp.int32)
            yi = y.astype(jnp.int32)
            x2 = jnp.minimum(xi + 1, top)
            y2 = jnp.minimum(yi + 1, top)
            sl = pl.ds(i * 16, 16)
            idx_v[sl] = xi * 2
            idx_v[pl.ds(C + i * 16, 16)] = x2 * 2
            idx_v[pl.ds(2 * C + i * 16, 16)] = yi * 2 + 1
            idx_v[pl.ds(3 * C + i * 16, 16)] = y2 * 2 + 1
            w_v[sl] = x2.astype(jnp.float32) - x
            w_v[pl.ds(C + i * 16, 16)] = x - xi.astype(jnp.float32)
            w_v[pl.ds(2 * C + i * 16, 16)] = y2.astype(jnp.float32) - y
            w_v[pl.ds(3 * C + i * 16, 16)] = y - yi.astype(jnp.float32)
            return carry2

        lax.fori_loop(0, C // 16, idx_body, 0)

        pltpu.async_copy(cb_hbm.at[idx_v], feat_v, sem).wait()

        def comb_body(c, carry2):
            cs = jnp.full((16,), c, jnp.int32)
            w1 = plsc.load_gather(w_v, [cs])
            w2 = plsc.load_gather(w_v, [cs + C])
            w3 = plsc.load_gather(w_v, [cs + 2 * C])
            w4 = plsc.load_gather(w_v, [cs + 3 * C])
            for s in range(SEG):
                sl = pl.ds(s * 16, 16)
                a = feat_v[c, sl]
                b = feat_v[C + c, sl]
                cc = feat_v[2 * C + c, sl]
                d = feat_v[3 * C + c, sl]
                feat_v[c, sl] = (w1 * a + w2 * b) + (w3 * cc + w4 * d)
            return carry2

        lax.fori_loop(0, C, comb_body, 0)

        pltpu.sync_copy(feat_v.at[pl.ds(0, C)], out_hbm.at[pl.ds(base, C)])
        return carry

    lax.fori_loop(0, nch_w, chunk_body, 0)


@jax.jit
def kernel(pts, codebook_0):
    cb2 = codebook_0.reshape(2 * R, F)
    pts_flat = pts.reshape(2 * N)
    run = functools.partial(
        pl.kernel,
        mesh=plsc.VectorSubcoreMesh(core_axis_name="c", subcore_axis_name="s"),
        out_type=jax.ShapeDtypeStruct((N, F), jnp.float32),
        scratch_types=[
            pltpu.VMEM((2 * C,), jnp.float32),
            pltpu.VMEM((4 * C,), jnp.int32),
            pltpu.VMEM((4 * C,), jnp.float32),
            pltpu.VMEM((4 * C, F), jnp.float32),
            pltpu.SemaphoreType.DMA,
        ],
        compiler_params=pltpu.CompilerParams(
            needs_layout_passes=False, use_tc_tiling_on_sc=False
        ),
    )(_body)
    return run(pts_flat, cb2)


# speedup vs baseline: 2.4202x; 2.0936x over previous
import functools

import jax
import jax.numpy as jnp
from jax import lax
from jax.experimental import pallas as pl
from jax.experimental.pallas import tpu as pltpu
from jax.experimental.pallas import tpu_sc as plsc

R = 2 ** 18
F = 48
N = 1000000
C = 160
NCH = N // C
NW = 32
SEG = F // 16
NG = C // 16


def _body(px_hbm, py_hbm, cb_hbm, out_hbm,
          px_v, py_v, idx_a, idx_b, w_a, w_b, feat_a, feat_b,
          sem_a, sem_b):
    wid = lax.axis_index("s") * 2 + lax.axis_index("c")
    nch_w = (NCH - wid + NW - 1) // NW
    lanes = jnp.arange(16, dtype=jnp.int32)
    lim = jnp.float32(R - 1 - 1e-5)
    top = jnp.full((16,), R - 1, jnp.int32)

    def stage_idx(cid, idx_v, w_v):
        base = cid * C
        pltpu.sync_copy(px_hbm.at[pl.ds(base, C)], px_v)
        pltpu.sync_copy(py_hbm.at[pl.ds(base, C)], py_v)

        def idx_body(i, carry):
            sl = pl.ds(i * 16, 16)
            x = jnp.maximum(jnp.minimum(px_v[sl] * (R - 1.0), lim), 0.0)
            y = jnp.maximum(jnp.minimum(py_v[sl] * (R - 1.0), lim), 0.0)
            xi = x.astype(jnp.int32)
            yi = y.astype(jnp.int32)
            x2 = jnp.minimum(xi + 1, top)
            y2 = jnp.minimum(yi + 1, top)
            idx_v[sl] = xi * 2
            idx_v[pl.ds(C + i * 16, 16)] = x2 * 2
            idx_v[pl.ds(2 * C + i * 16, 16)] = yi * 2 + 1
            idx_v[pl.ds(3 * C + i * 16, 16)] = y2 * 2 + 1
            w_v[sl] = x2.astype(jnp.float32) - x
            w_v[pl.ds(C + i * 16, 16)] = x - xi.astype(jnp.float32)
            w_v[pl.ds(2 * C + i * 16, 16)] = y2.astype(jnp.float32) - y
            w_v[pl.ds(3 * C + i * 16, 16)] = y - yi.astype(jnp.float32)
            return carry

        lax.fori_loop(0, NG, idx_body, 0)

    def stage_combine_out(cid, w_v, feat_v):
        def comb_body(g, carry):
            w1g = w_v[pl.ds(g * 16, 16)]
            w2g = w_v[pl.ds(C + g * 16, 16)]
            w3g = w_v[pl.ds(2 * C + g * 16, 16)]
            w4g = w_v[pl.ds(3 * C + g * 16, 16)]
            for l in range(16):
                c = g * 16 + l
                w1 = w1g[l]
                w2 = w2g[l]
                w3 = w3g[l]
                w4 = w4g[l]
                for s in range(SEG):
                    sl = pl.ds(s * 16, 16)
                    a = feat_v[c, sl]
                    b = feat_v[C + c, sl]
                    cc = feat_v[2 * C + c, sl]
                    d = feat_v[3 * C + c, sl]
                    feat_v[c, sl] = (w1 * a + w2 * b) + (w3 * cc + w4 * d)
            return carry

        lax.fori_loop(0, NG, comb_body, 0)
        pltpu.sync_copy(feat_v.at[pl.ds(0, C)],
                        out_hbm.at[pl.ds(cid * C, C)])

    def gather_start(idx_v, feat_v, sem):
        return pltpu.async_copy(cb_hbm.at[idx_v], feat_v, sem)

    stage_idx(wid, idx_a, w_a)
    gather_start(idx_a, feat_a, sem_a)

    npairs = (nch_w + 1) // 2

    def pair_body(j, carry):
        ka = 2 * j
        kb = 2 * j + 1
        cid_a = wid + ka * NW
        cid_b = wid + kb * NW

        @pl.when(kb < nch_w)
        def _():
            stage_idx(cid_b, idx_b, w_b)
            gather_start(idx_b, feat_b, sem_b)

        pltpu.make_async_copy(cb_hbm.at[idx_a], feat_a, sem_a).wait()
        stage_combine_out(cid_a, w_a, feat_a)

        @pl.when(kb < nch_w)
        def _():
            @pl.when(kb + 1 < nch_w)
            def _():
                stage_idx(cid_b + NW, idx_a, w_a)
                gather_start(idx_a, feat_a, sem_a)

            pltpu.make_async_copy(cb_hbm.at[idx_b], feat_b, sem_b).wait()
            stage_combine_out(cid_b, w_b, feat_b)

        return carry

    lax.fori_loop(0, npairs, pair_body, 0)


@jax.jit
def kernel(pts, codebook_0):
    cb2 = codebook_0.reshape(2 * R, F)
    px = pts[:, 0]
    py = pts[:, 1]
    run = functools.partial(
        pl.kernel,
        mesh=plsc.VectorSubcoreMesh(core_axis_name="c", subcore_axis_name="s"),
        out_type=jax.ShapeDtypeStruct((N, F), jnp.float32),
        scratch_types=[
            pltpu.VMEM((C,), jnp.float32),
            pltpu.VMEM((C,), jnp.float32),
            pltpu.VMEM((4 * C,), jnp.int32),
            pltpu.VMEM((4 * C,), jnp.int32),
            pltpu.VMEM((4 * C,), jnp.float32),
            pltpu.VMEM((4 * C,), jnp.float32),
            pltpu.VMEM((4 * C, F), jnp.float32),
            pltpu.VMEM((4 * C, F), jnp.float32),
            pltpu.SemaphoreType.DMA,
            pltpu.SemaphoreType.DMA,
        ],
        compiler_params=pltpu.CompilerParams(
            needs_layout_passes=False, use_tc_tiling_on_sc=False
        ),
    )(_body)
    return run(px, py, cb2)


# speedup vs baseline: 2.5763x; 1.0645x over previous
import functools

import jax
import jax.numpy as jnp
from jax import lax
from jax.experimental import pallas as pl
from jax.experimental.pallas import tpu as pltpu
from jax.experimental.pallas import tpu_sc as plsc

R = 2 ** 18
F = 48
N = 1000000
C = 320
NCH = N // C
NW = 32
SEG = F // 16
NG = C // 16


def _body(px_hbm, py_hbm, cb_hbm, out_hbm,
          px_v, py_v, idx_a, idx_b, w_a, w_b, feat_a, feat_b,
          sem_a, sem_b):
    wid = lax.axis_index("s") * 2 + lax.axis_index("c")
    nch_w = (NCH - wid + NW - 1) // NW
    lanes = jnp.arange(16, dtype=jnp.int32)
    lim = jnp.float32(R - 1 - 1e-5)
    top = jnp.full((16,), R - 1, jnp.int32)

    def stage_idx(cid, idx_v, w_v):
        base = cid * C
        pltpu.sync_copy(px_hbm.at[pl.ds(base, C)], px_v)
        pltpu.sync_copy(py_hbm.at[pl.ds(base, C)], py_v)

        def idx_body(i, carry):
            sl = pl.ds(i * 16, 16)
            x = jnp.maximum(jnp.minimum(px_v[sl] * (R - 1.0), lim), 0.0)
            y = jnp.maximum(jnp.minimum(py_v[sl] * (R - 1.0), lim), 0.0)
            xi = x.astype(jnp.int32)
            yi = y.astype(jnp.int32)
            x2 = jnp.minimum(xi + 1, top)
            y2 = jnp.minimum(yi + 1, top)
            idx_v[sl] = xi * 2
            idx_v[pl.ds(C + i * 16, 16)] = x2 * 2
            idx_v[pl.ds(2 * C + i * 16, 16)] = yi * 2 + 1
            idx_v[pl.ds(3 * C + i * 16, 16)] = y2 * 2 + 1
            w_v[sl] = x2.astype(jnp.float32) - x
            w_v[pl.ds(C + i * 16, 16)] = x - xi.astype(jnp.float32)
            w_v[pl.ds(2 * C + i * 16, 16)] = y2.astype(jnp.float32) - y
            w_v[pl.ds(3 * C + i * 16, 16)] = y - yi.astype(jnp.float32)
            return carry

        lax.fori_loop(0, NG, idx_body, 0)

    def stage_combine_out(cid, w_v, feat_v):
        def comb_body(g, carry):
            w1g = w_v[pl.ds(g * 16, 16)]
            w2g = w_v[pl.ds(C + g * 16, 16)]
            w3g = w_v[pl.ds(2 * C + g * 16, 16)]
            w4g = w_v[pl.ds(3 * C + g * 16, 16)]
            for l in range(16):
                c = g * 16 + l
                w1 = w1g[l]
                w2 = w2g[l]
                w3 = w3g[l]
                w4 = w4g[l]
                for s in range(SEG):
                    sl = pl.ds(s * 16, 16)
                    a = feat_v[c, sl]
                    b = feat_v[C + c, sl]
                    cc = feat_v[2 * C + c, sl]
                    d = feat_v[3 * C + c, sl]
                    feat_v[c, sl] = (w1 * a + w2 * b) + (w3 * cc + w4 * d)
            return carry

        lax.fori_loop(0, NG, comb_body, 0)
        pltpu.sync_copy(feat_v.at[pl.ds(0, C)],
                        out_hbm.at[pl.ds(cid * C, C)])

    def gather_start(idx_v, feat_v, sem):
        return pltpu.async_copy(cb_hbm.at[idx_v], feat_v, sem)

    stage_idx(wid, idx_a, w_a)
    gather_start(idx_a, feat_a, sem_a)

    npairs = (nch_w + 1) // 2

    def pair_body(j, carry):
        ka = 2 * j
        kb = 2 * j + 1
        cid_a = wid + ka * NW
        cid_b = wid + kb * NW

        @pl.when(kb < nch_w)
        def _():
            stage_idx(cid_b, idx_b, w_b)
            gather_start(idx_b, feat_b, sem_b)

        pltpu.make_async_copy(cb_hbm.at[idx_a], feat_a, sem_a).wait()
        stage_combine_out(cid_a, w_a, feat_a)

        @pl.when(kb < nch_w)
        def _():
            @pl.when(kb + 1 < nch_w)
            def _():
                stage_idx(cid_b + NW, idx_a, w_a)
                gather_start(idx_a, feat_a, sem_a)

            pltpu.make_async_copy(cb_hbm.at[idx_b], feat_b, sem_b).wait()
            stage_combine_out(cid_b, w_b, feat_b)

        return carry

    lax.fori_loop(0, npairs, pair_body, 0)


@jax.jit
def kernel(pts, codebook_0):
    cb2 = codebook_0.reshape(2 * R, F)
    px = pts[:, 0]
    py = pts[:, 1]
    run = functools.partial(
        pl.kernel,
        mesh=plsc.VectorSubcoreMesh(core_axis_name="c", subcore_axis_name="s"),
        out_type=jax.ShapeDtypeStruct((N, F), jnp.float32),
        scratch_types=[
            pltpu.VMEM((C,), jnp.float32),
            pltpu.VMEM((C,), jnp.float32),
            pltpu.VMEM((4 * C,), jnp.int32),
            pltpu.VMEM((4 * C,), jnp.int32),
            pltpu.VMEM((4 * C,), jnp.float32),
            pltpu.VMEM((4 * C,), jnp.float32),
            pltpu.VMEM((4 * C, F), jnp.float32),
            pltpu.VMEM((4 * C, F), jnp.float32),
            pltpu.SemaphoreType.DMA,
            pltpu.SemaphoreType.DMA,
        ],
        compiler_params=pltpu.CompilerParams(
            needs_layout_passes=False, use_tc_tiling_on_sc=False
        ),
    )(_body)
    return run(px, py, cb2)
